# baseline (speedup 1.0000x reference)
import functools

import jax
import jax.numpy as jnp
from jax import lax
from jax.experimental import pallas as pl
from jax.experimental.pallas import tpu as pltpu
from jax.experimental.pallas import tpu_sc as plsc

ENC = 128
HID = 64
K = 10000
BLK = 1600
WIN = BLK + 8
K_PAD = K + WIN


def _phi_seg_kernel(inv_ref, x_ref, w1t_ref, b1_ref, w2t_ref, b2_ref,
                    acc_ref):
    i = pl.program_id(0)

    @pl.when(i == 0)
    def _init():
        acc_ref[...] = jnp.zeros_like(acc_ref)

    x = x_ref[...]
    z = jnp.dot(x, w1t_ref[...], preferred_element_type=jnp.float32)
    z = z + b1_ref[...]
    z = z * jax.nn.sigmoid(z)
    z = jnp.dot(z, w2t_ref[...], preferred_element_type=jnp.float32)
    z = z + b2_ref[...]
    z = z * jax.nn.sigmoid(z)

    inv = inv_ref[0, 0, :]
    base = (inv_ref[0, 0, 0] // 8) * 8
    base = pl.multiple_of(base, 8)
    r = inv - base
    hit = lax.broadcasted_iota(jnp.int32, (WIN, BLK), 0) == r[None, :]
    oh = hit.astype(jnp.bfloat16)
    z_ext = jnp.concatenate([z, jnp.ones((BLK, 1), jnp.float32)], axis=1)
    partial = jnp.dot(oh, z_ext.astype(jnp.bfloat16),
                      preferred_element_type=jnp.float32)
    acc_ref[pl.ds(base, WIN), :] += partial


def _rho_kernel(acc_ref, cnt_ref, w3t_ref, b3_ref, h_ref):
    mean = acc_ref[...] / cnt_ref[...]
    h = jnp.dot(mean, w3t_ref[...], preferred_element_type=jnp.float32)
    h = h + b3_ref[...]
    h_ref[...] = h * jax.nn.sigmoid(h)


_NC = 2
_NS = 16
_NW = _NC * _NS


_CH = 80
_KDMA = 25


def _sc_gather(table2, idx2):
    nw, per_w_rows, one, ch = idx2.shape
    per_w = per_w_rows * ch
    rounds = per_w_rows // _KDMA
    assert per_w_rows % _KDMA == 0 and ch == _CH and nw == _NW and one == 1
    mesh = plsc.VectorSubcoreMesh(core_axis_name="c", subcore_axis_name="s",
                                  num_cores=_NC, num_subcores=_NS)

    @functools.partial(
        pl.kernel, mesh=mesh,
        out_type=jax.ShapeDtypeStruct((_NW, per_w, 16), jnp.float32),
        scratch_types=[
            pltpu.VMEM((per_w_rows, 1, _CH), jnp.int32),
            pltpu.VMEM((_KDMA, _CH, 16), jnp.float32),
            pltpu.SemaphoreType.DMA,
            pltpu.SemaphoreType.DMA,
        ],
        compiler_params=pltpu.CompilerParams(use_tc_tiling_on_sc=False),
    )
    def gather_kernel(table_hbm, idx_hbm, out_hbm, idx_v, rows_v,
                      gsem, ssem):
        wid = lax.axis_index("s") * _NC + lax.axis_index("c")
        pltpu.sync_copy(idx_hbm.at[wid], idx_v)
        my_out = out_hbm.at[wid]

        @pl.loop(0, rounds)
        def _round(r):
            c0 = r * _KDMA
            for b in range(_KDMA):
                pltpu.async_copy(table_hbm.at[idx_v.at[c0 + b, 0]],
                                 rows_v.at[b], gsem)
            for b in range(_KDMA):
                pltpu.make_async_copy(table_hbm.at[idx_v.at[c0 + b, 0]],
                                      rows_v.at[b], gsem).wait()
            for b in range(_KDMA):
                dst = my_out.at[pl.ds((c0 + b) * _CH, _CH), :]
                pltpu.async_copy(rows_v.at[b], dst, ssem)
            for b in range(_KDMA):
                dst = my_out.at[pl.ds((c0 + b) * _CH, _CH), :]
                pltpu.make_async_copy(rows_v.at[b], dst, ssem).wait()

    return gather_kernel(table2, idx2)


def kernel(x, group_labels, W1, b1, W2, b2, W3, b3, Wa, ba, Wb, bb):
    B = x.shape[0]
    nb = B // BLK

    labels = group_labels.astype(jnp.int32)
    is_new = jnp.concatenate(
        [jnp.zeros((1,), jnp.int32),
         (labels[1:] != labels[:-1]).astype(jnp.int32)])
    inverse = jnp.cumsum(is_new)
    inv3 = inverse.reshape(nb, 1, BLK)

    acc = pl.pallas_call(
        _phi_seg_kernel,
        grid=(nb,),
        in_specs=[
            pl.BlockSpec((1, 1, BLK), lambda i: (i, 0, 0)),
            pl.BlockSpec((BLK, ENC), lambda i: (i, 0)),
            pl.BlockSpec((ENC, HID), lambda i: (0, 0)),
            pl.BlockSpec((1, HID), lambda i: (0, 0)),
            pl.BlockSpec((HID, HID), lambda i: (0, 0)),
            pl.BlockSpec((1, HID), lambda i: (0, 0)),
        ],
        out_specs=pl.BlockSpec((K_PAD, HID + 1), lambda i: (0, 0)),
        out_shape=jax.ShapeDtypeStruct((K_PAD, HID + 1), jnp.float32),
    )(inv3, x, W1.T, b1[None, :], W2.T, b2[None, :])

    h = pl.pallas_call(
        _rho_kernel,
        in_specs=[
            pl.BlockSpec((K, HID), lambda: (0, 0)),
            pl.BlockSpec((K, 1), lambda: (0, 0)),
            pl.BlockSpec((HID, HID), lambda: (0, 0)),
            pl.BlockSpec((1, HID), lambda: (0, 0)),
        ],
        out_specs=pl.BlockSpec((K, HID), lambda: (0, 0)),
        out_shape=jax.ShapeDtypeStruct((K, HID), jnp.float32),
    )(acc[:K, :HID], acc[:K, HID:], W3.T, b3[None, :])

    alpha = jax.nn.softplus(h @ Wa.T + ba)[:, 0] + 1e-4
    beta = jax.nn.softplus(h @ Wb.T + bb)[:, 0] + 1e-4
    g = jax.random.gamma(jax.random.key(1), alpha)
    tau_group = g / beta
    table2 = jnp.broadcast_to(tau_group[:, None], (K, 16))
    idx2 = inverse.reshape(_NW, (B // _NW) // _CH, 1, _CH)
    tau_per_refl = _sc_gather(table2, idx2).reshape(B, 16)[:, :1]
    return alpha, beta, tau_per_refl


# speedup vs baseline: 1.0928x; 1.0928x over previous
import functools

import jax
import jax.numpy as jnp
from jax import lax
from jax.experimental import pallas as pl
from jax.experimental.pallas import tpu as pltpu
from jax.experimental.pallas import tpu_sc as plsc

ENC = 128
HID = 64
K = 10000
BLK = 1000
WIN = BLK + 8
K_PAD = K + WIN


def _phi_seg_kernel(inv_ref, x_ref, w1t_ref, b1_ref, w2t_ref, b2_ref,
                    acc_ref):
    i = pl.program_id(0)

    @pl.when(i == 0)
    def _init():
        acc_ref[...] = jnp.zeros_like(acc_ref)

    x = x_ref[...]
    z = jnp.dot(x, w1t_ref[...], preferred_element_type=jnp.float32)
    z = z + b1_ref[...]
    z = z * jax.nn.sigmoid(z)
    z = jnp.dot(z, w2t_ref[...], preferred_element_type=jnp.float32)
    z = z + b2_ref[...]
    z = z * jax.nn.sigmoid(z)

    inv = inv_ref[0, 0, :]
    base = (inv_ref[0, 0, 0] // 8) * 8
    base = pl.multiple_of(base, 8)
    r = inv - base
    hit = lax.broadcasted_iota(jnp.int32, (WIN, BLK), 0) == r[None, :]
    oh = hit.astype(jnp.bfloat16)
    z_ext = jnp.concatenate([z, jnp.ones((BLK, 1), jnp.float32)], axis=1)
    partial = jnp.dot(oh, z_ext.astype(jnp.bfloat16),
                      preferred_element_type=jnp.float32)
    acc_ref[pl.ds(base, WIN), :] += partial


def _rho_kernel(acc_ref, cnt_ref, w3t_ref, b3_ref, h_ref):
    mean = acc_ref[...] / cnt_ref[...]
    h = jnp.dot(mean, w3t_ref[...], preferred_element_type=jnp.float32)
    h = h + b3_ref[...]
    h_ref[...] = h * jax.nn.sigmoid(h)


_NC = 2
_NS = 16
_NW = _NC * _NS


_CH = 80
_KDMA = 25


def _sc_gather(table2, idx2):
    nw, per_w_rows, one, ch = idx2.shape
    per_w = per_w_rows * ch
    rounds = per_w_rows // _KDMA
    assert per_w_rows % _KDMA == 0 and ch == _CH and nw == _NW and one == 1
    mesh = plsc.VectorSubcoreMesh(core_axis_name="c", subcore_axis_name="s",
                                  num_cores=_NC, num_subcores=_NS)

    @functools.partial(
        pl.kernel, mesh=mesh,
        out_type=jax.ShapeDtypeStruct((_NW, per_w, 16), jnp.float32),
        scratch_types=[
            pltpu.VMEM((per_w_rows, 1, _CH), jnp.int32),
            pltpu.VMEM((_KDMA, _CH, 16), jnp.float32),
            pltpu.SemaphoreType.DMA,
            pltpu.SemaphoreType.DMA,
        ],
        compiler_params=pltpu.CompilerParams(use_tc_tiling_on_sc=False),
    )
    def gather_kernel(table_hbm, idx_hbm, out_hbm, idx_v, rows_v,
                      gsem, ssem):
        wid = lax.axis_index("s") * _NC + lax.axis_index("c")
        pltpu.sync_copy(idx_hbm.at[wid], idx_v)
        my_out = out_hbm.at[wid]

        @pl.loop(0, rounds)
        def _round(r):
            c0 = r * _KDMA
            for b in range(_KDMA):
                pltpu.async_copy(table_hbm.at[idx_v.at[c0 + b, 0]],
                                 rows_v.at[b], gsem)
            for b in range(_KDMA):
                pltpu.make_async_copy(table_hbm.at[idx_v.at[c0 + b, 0]],
                                      rows_v.at[b], gsem).wait()
            for b in range(_KDMA):
                dst = my_out.at[pl.ds((c0 + b) * _CH, _CH), :]
                pltpu.async_copy(rows_v.at[b], dst, ssem)
            for b in range(_KDMA):
                dst = my_out.at[pl.ds((c0 + b) * _CH, _CH), :]
                pltpu.make_async_copy(rows_v.at[b], dst, ssem).wait()

    return gather_kernel(table2, idx2)


def kernel(x, group_labels, W1, b1, W2, b2, W3, b3, Wa, ba, Wb, bb):
    B = x.shape[0]
    nb = B // BLK

    labels = group_labels.astype(jnp.int32)
    is_new = jnp.concatenate(
        [jnp.zeros((1,), jnp.int32),
         (labels[1:] != labels[:-1]).astype(jnp.int32)])
    inverse = jnp.cumsum(is_new)
    inv3 = inverse.reshape(nb, 1, BLK)

    acc = pl.pallas_call(
        _phi_seg_kernel,
        grid=(nb,),
        in_specs=[
            pl.BlockSpec((1, 1, BLK), lambda i: (i, 0, 0)),
            pl.BlockSpec((BLK, ENC), lambda i: (i, 0)),
            pl.BlockSpec((ENC, HID), lambda i: (0, 0)),
            pl.BlockSpec((1, HID), lambda i: (0, 0)),
            pl.BlockSpec((HID, HID), lambda i: (0, 0)),
            pl.BlockSpec((1, HID), lambda i: (0, 0)),
        ],
        out_specs=pl.BlockSpec((K_PAD, HID + 1), lambda i: (0, 0)),
        out_shape=jax.ShapeDtypeStruct((K_PAD, HID + 1), jnp.float32),
    )(inv3, x, W1.T, b1[None, :], W2.T, b2[None, :])

    h = pl.pallas_call(
        _rho_kernel,
        in_specs=[
            pl.BlockSpec((K, HID), lambda: (0, 0)),
            pl.BlockSpec((K, 1), lambda: (0, 0)),
            pl.BlockSpec((HID, HID), lambda: (0, 0)),
            pl.BlockSpec((1, HID), lambda: (0, 0)),
        ],
        out_specs=pl.BlockSpec((K, HID), lambda: (0, 0)),
        out_shape=jax.ShapeDtypeStruct((K, HID), jnp.float32),
    )(acc[:K, :HID], acc[:K, HID:], W3.T, b3[None, :])

    alpha = jax.nn.softplus(h @ Wa.T + ba)[:, 0] + 1e-4
    beta = jax.nn.softplus(h @ Wb.T + bb)[:, 0] + 1e-4
    g = jax.random.gamma(jax.random.key(1), alpha)
    tau_group = g / beta
    table2 = jnp.broadcast_to(tau_group[:, None], (K, 16))
    idx2 = inverse.reshape(_NW, (B // _NW) // _CH, 1, _CH)
    tau_per_refl = _sc_gather(table2, idx2).reshape(B, 16)[:, :1]
    return alpha, beta, tau_per_refl


# speedup vs baseline: 1.0935x; 1.0007x over previous
import functools

import jax
import jax.numpy as jnp
from jax import lax
from jax.experimental import pallas as pl
from jax.experimental.pallas import tpu as pltpu
from jax.experimental.pallas import tpu_sc as plsc

ENC = 128
HID = 64
K = 10000
BLK = 1280
WIN = BLK + 8
K_PAD = K + WIN


def _phi_seg_kernel(inv_ref, x_ref, w1t_ref, b1_ref, w2t_ref, b2_ref,
                    acc_ref):
    i = pl.program_id(0)

    @pl.when(i == 0)
    def _init():
        acc_ref[...] = jnp.zeros_like(acc_ref)

    x = x_ref[...]
    z = jnp.dot(x, w1t_ref[...], preferred_element_type=jnp.float32)
    z = z + b1_ref[...]
    z = z * jax.nn.sigmoid(z)
    z = jnp.dot(z, w2t_ref[...], preferred_element_type=jnp.float32)
    z = z + b2_ref[...]
    z = z * jax.nn.sigmoid(z)

    inv = inv_ref[0, 0, :]
    base = (inv_ref[0, 0, 0] // 8) * 8
    base = pl.multiple_of(base, 8)
    r = inv - base
    hit = lax.broadcasted_iota(jnp.int32, (WIN, BLK), 0) == r[None, :]
    oh = hit.astype(jnp.bfloat16)
    z_ext = jnp.concatenate([z, jnp.ones((BLK, 1), jnp.float32)], axis=1)
    partial = jnp.dot(oh, z_ext.astype(jnp.bfloat16),
                      preferred_element_type=jnp.float32)
    acc_ref[pl.ds(base, WIN), :] += partial


def _rho_kernel(acc_ref, cnt_ref, w3t_ref, b3_ref, h_ref):
    mean = acc_ref[...] / cnt_ref[...]
    h = jnp.dot(mean, w3t_ref[...], preferred_element_type=jnp.float32)
    h = h + b3_ref[...]
    h_ref[...] = h * jax.nn.sigmoid(h)


_NC = 2
_NS = 16
_NW = _NC * _NS


_CH = 80
_KDMA = 25


def _sc_gather(table2, idx2):
    nw, per_w_rows, one, ch = idx2.shape
    per_w = per_w_rows * ch
    rounds = per_w_rows // _KDMA
    assert per_w_rows % _KDMA == 0 and ch == _CH and nw == _NW and one == 1
    mesh = plsc.VectorSubcoreMesh(core_axis_name="c", subcore_axis_name="s",
                                  num_cores=_NC, num_subcores=_NS)

    @functools.partial(
        pl.kernel, mesh=mesh,
        out_type=jax.ShapeDtypeStruct((_NW, per_w, 16), jnp.float32),
        scratch_types=[
            pltpu.VMEM((per_w_rows, 1, _CH), jnp.int32),
            pltpu.VMEM((_KDMA, _CH, 16), jnp.float32),
            pltpu.SemaphoreType.DMA,
            pltpu.SemaphoreType.DMA,
        ],
        compiler_params=pltpu.CompilerParams(use_tc_tiling_on_sc=False),
    )
    def gather_kernel(table_hbm, idx_hbm, out_hbm, idx_v, rows_v,
                      gsem, ssem):
        wid = lax.axis_index("s") * _NC + lax.axis_index("c")
        pltpu.sync_copy(idx_hbm.at[wid], idx_v)
        my_out = out_hbm.at[wid]

        @pl.loop(0, rounds)
        def _round(r):
            c0 = r * _KDMA
            for b in range(_KDMA):
                pltpu.async_copy(table_hbm.at[idx_v.at[c0 + b, 0]],
                                 rows_v.at[b], gsem)
            for b in range(_KDMA):
                pltpu.make_async_copy(table_hbm.at[idx_v.at[c0 + b, 0]],
                                      rows_v.at[b], gsem).wait()
            for b in range(_KDMA):
                dst = my_out.at[pl.ds((c0 + b) * _CH, _CH), :]
                pltpu.async_copy(rows_v.at[b], dst, ssem)
            for b in range(_KDMA):
                dst = my_out.at[pl.ds((c0 + b) * _CH, _CH), :]
                pltpu.make_async_copy(rows_v.at[b], dst, ssem).wait()

    return gather_kernel(table2, idx2)


def kernel(x, group_labels, W1, b1, W2, b2, W3, b3, Wa, ba, Wb, bb):
    B = x.shape[0]
    nb = B // BLK

    labels = group_labels.astype(jnp.int32)
    is_new = jnp.concatenate(
        [jnp.zeros((1,), jnp.int32),
         (labels[1:] != labels[:-1]).astype(jnp.int32)])
    inverse = jnp.cumsum(is_new)
    inv3 = inverse.reshape(nb, 1, BLK)

    acc = pl.pallas_call(
        _phi_seg_kernel,
        grid=(nb,),
        in_specs=[
            pl.BlockSpec((1, 1, BLK), lambda i: (i, 0, 0)),
            pl.BlockSpec((BLK, ENC), lambda i: (i, 0)),
            pl.BlockSpec((ENC, HID), lambda i: (0, 0)),
            pl.BlockSpec((1, HID), lambda i: (0, 0)),
            pl.BlockSpec((HID, HID), lambda i: (0, 0)),
            pl.BlockSpec((1, HID), lambda i: (0, 0)),
        ],
        out_specs=pl.BlockSpec((K_PAD, HID + 1), lambda i: (0, 0)),
        out_shape=jax.ShapeDtypeStruct((K_PAD, HID + 1), jnp.float32),
    )(inv3, x, W1.T, b1[None, :], W2.T, b2[None, :])

    h = pl.pallas_call(
        _rho_kernel,
        in_specs=[
            pl.BlockSpec((K, HID), lambda: (0, 0)),
            pl.BlockSpec((K, 1), lambda: (0, 0)),
            pl.BlockSpec((HID, HID), lambda: (0, 0)),
            pl.BlockSpec((1, HID), lambda: (0, 0)),
        ],
        out_specs=pl.BlockSpec((K, HID), lambda: (0, 0)),
        out_shape=jax.ShapeDtypeStruct((K, HID), jnp.float32),
    )(acc[:K, :HID], acc[:K, HID:], W3.T, b3[None, :])

    alpha = jax.nn.softplus(h @ Wa.T + ba)[:, 0] + 1e-4
    beta = jax.nn.softplus(h @ Wb.T + bb)[:, 0] + 1e-4
    g = jax.random.gamma(jax.random.key(1), alpha)
    tau_group = g / beta
    table2 = jnp.broadcast_to(tau_group[:, None], (K, 16))
    idx2 = inverse.reshape(_NW, (B // _NW) // _CH, 1, _CH)
    tau_per_refl = _sc_gather(table2, idx2).reshape(B, 16)[:, :1]
    return alpha, beta, tau_per_refl
